# Initial kernel scaffold; baseline (speedup 1.0000x reference)
import jax
import jax.numpy as jnp
from jax.experimental import pallas as pl


def kernel(obj_heads, reg_heads, cls_heads, batch_anchors):
    raise NotImplementedError("write your pallas kernel here")


# speedup vs baseline: 9.0988x; 9.0988x over previous
import functools

import jax
import jax.numpy as jnp
from jax.experimental import pallas as pl
from jax.experimental.pallas import tpu as pltpu

IMAGE_W = 608
IMAGE_H = 608
TOP_N = 1000
MIN_SCORE = 0.05
NMS_THR = 0.5
MAX_DET = 100

_CHUNK = 2048


def _sig(x):
    return 1.0 / (1.0 + jnp.exp(-x))


def _decode_body(obj_ref, reg_ref, cls_ref, anch_ref, score_ref, rest_ref):
    o = obj_ref[0, 0, :, 0]
    creg = reg_ref[0, 0]
    ccls = cls_ref[0, 0]
    a = anch_ref[0, 0]
    m = jnp.max(ccls, axis=-1)
    amax = jnp.argmax(ccls, axis=-1).astype(jnp.float32)
    score = _sig(m) * _sig(o)
    stride = a[:, 4]
    xyx = (_sig(creg[:, 0]) + a[:, 0]) * stride
    xyy = (_sig(creg[:, 1]) + a[:, 1]) * stride
    whx = jnp.exp(creg[:, 2]) * a[:, 2] / stride
    why = jnp.exp(creg[:, 3]) * a[:, 3] / stride
    x1f = xyx - 0.5 * whx
    y1f = xyy - 0.5 * why
    x2f = whx + x1f
    y2f = why + y1f
    x1 = jnp.maximum(x1f.astype(jnp.int32), 0).astype(jnp.float32)
    y1 = jnp.maximum(y1f.astype(jnp.int32), 0).astype(jnp.float32)
    x2 = jnp.minimum(x2f.astype(jnp.int32), IMAGE_W - 1).astype(jnp.float32)
    y2 = jnp.minimum(y2f.astype(jnp.int32), IMAGE_H - 1).astype(jnp.float32)
    score_ref[0, 0, 0, :] = score
    rest_ref[0, 0, 0, 0, :] = amax
    rest_ref[1, 0, 0, 0, :] = x1
    rest_ref[2, 0, 0, 0, :] = y1
    rest_ref[3, 0, 0, 0, :] = x2
    rest_ref[4, 0, 0, 0, :] = y2


def _threshold_body(score_ref, eff_ref):
    score = score_ref[...]
    B, L, N = score.shape
    bits = jax.lax.bitcast_convert_type(score, jnp.int32)
    idx = jax.lax.broadcasted_iota(jnp.int32, (B, L, N), 2)

    def bit_step(_, carry):
        lo, hi = carry
        mid = jax.lax.shift_right_logical(lo + hi + 1, 1)
        cnt = jnp.sum((bits >= mid[:, :, None]).astype(jnp.int32), axis=-1)
        ok = cnt >= TOP_N
        return jnp.where(ok, mid, lo), jnp.where(ok, hi, mid - 1)

    lo0 = jnp.zeros((B, L), jnp.int32)
    hi0 = jnp.full((B, L), 0x3F800000, jnp.int32)
    t, _ = jax.lax.fori_loop(0, 31, bit_step, (lo0, hi0))
    gt = bits > t[:, :, None]
    eq = bits == t[:, :, None]
    need = TOP_N - jnp.sum(gt.astype(jnp.int32), axis=-1)

    def idx_step(_, carry):
        lo, hi = carry
        mid = jax.lax.shift_right_logical(lo + hi, 1)
        cnt = jnp.sum((eq & (idx < mid[:, :, None])).astype(jnp.int32), axis=-1)
        ok = cnt >= need
        return jnp.where(ok, lo, mid + 1), jnp.where(ok, mid, hi)

    lo0i = jnp.zeros((B, L), jnp.int32)
    hi0i = jnp.full((B, L), N, jnp.int32)
    _, icut = jax.lax.fori_loop(0, 15, idx_step, (lo0i, hi0i))
    mask = gt | (eq & (idx < icut[:, :, None]))
    eff_ref[...] = jnp.where(mask & (score > MIN_SCORE), score, -1.0)


def _nms_body(rest_ref, eff_ref, s_ref, c_ref, b_ref, scr_ref, ha_ref):
    B, W = eff_ref.shape
    cls_p = rest_ref[0]
    x1 = rest_ref[1]
    y1 = rest_ref[2]
    x2 = rest_ref[3]
    y2 = rest_ref[4]
    scr_ref[...] = eff_ref[...]
    ha_ref[...] = 0.5 * ((x2 - x1) * (y2 - y1))
    lane = jax.lax.broadcasted_iota(jnp.int32, (B, W), 1)

    def step(k, _):
        masked = scr_ref[...]
        m = jnp.max(masked, axis=1)
        sel = jnp.argmax(masked, axis=1)
        found = m > 0.0
        onehot = lane == sel[:, None]
        neg = jnp.float32(-1e30)
        sx1 = jnp.max(jnp.where(onehot, x1, neg), axis=1)
        sy1 = jnp.max(jnp.where(onehot, y1, neg), axis=1)
        sx2 = jnp.max(jnp.where(onehot, x2, neg), axis=1)
        sy2 = jnp.max(jnp.where(onehot, y2, neg), axis=1)
        sc = jnp.max(jnp.where(onehot, cls_p, neg), axis=1)
        s_ref[pl.ds(k, 1), :] = jnp.where(found, m, -1.0)[None, :]
        c_ref[pl.ds(k, 1), :] = jnp.where(found, sc, -1.0)[None, :]
        box = jnp.stack([sx1, sy1, sx2, sy2], axis=-1)
        b_ref[pl.ds(k, 1), :, :] = jnp.where(found[:, None], box, -1.0)[None]
        sarea_eps = (sx2 - sx1) * (sy2 - sy1) + jnp.float32(1e-9)
        xx1 = jnp.maximum(x1, sx1[:, None])
        yy1 = jnp.maximum(y1, sy1[:, None])
        xx2 = jnp.minimum(x2, sx2[:, None])
        yy2 = jnp.minimum(y2, sy2[:, None])
        iw = jnp.maximum(xx2 - xx1, 0.0)
        ih = jnp.maximum(yy2 - yy1, 0.0)
        inter = iw * ih
        rhs = ha_ref[...] + (0.5 * sarea_eps)[:, None]
        sup = 1.5 * inter > rhs
        kill = (sup | onehot) & found[:, None]
        scr_ref[...] = jnp.where(kill, -1.0, masked)
        return 0

    jax.lax.fori_loop(0, MAX_DET, step, 0)


def kernel(obj_heads, reg_heads, cls_heads, batch_anchors):
    L, B, N, C = cls_heads.shape
    W = L * N
    nc = N // _CHUNK

    score, rest = pl.pallas_call(
        _decode_body,
        grid=(L, B, nc),
        in_specs=[
            pl.BlockSpec((1, 1, _CHUNK, 1), lambda l, b, c: (l, b, c, 0)),
            pl.BlockSpec((1, 1, _CHUNK, 4), lambda l, b, c: (l, b, c, 0)),
            pl.BlockSpec((1, 1, _CHUNK, C), lambda l, b, c: (l, b, c, 0)),
            pl.BlockSpec((1, 1, _CHUNK, 5), lambda l, b, c: (l, b, c, 0)),
        ],
        out_specs=[
            pl.BlockSpec((1, 1, 1, _CHUNK), lambda l, b, c: (b, l, 0, c)),
            pl.BlockSpec((5, 1, 1, 1, _CHUNK), lambda l, b, c: (0, b, l, 0, c)),
        ],
        out_shape=[
            jax.ShapeDtypeStruct((B, L, 1, N), jnp.float32),
            jax.ShapeDtypeStruct((5, B, L, 1, N), jnp.float32),
        ],
    )(obj_heads, reg_heads, cls_heads, batch_anchors)

    eff = pl.pallas_call(
        _threshold_body,
        out_shape=jax.ShapeDtypeStruct((B, L, N), jnp.float32),
    )(score.reshape(B, L, N))

    rest_f = rest.reshape(5, B, W)
    eff_f = eff.reshape(B, W)

    s_t, c_t, b_t = pl.pallas_call(
        _nms_body,
        out_shape=[
            jax.ShapeDtypeStruct((MAX_DET, B), jnp.float32),
            jax.ShapeDtypeStruct((MAX_DET, B), jnp.float32),
            jax.ShapeDtypeStruct((MAX_DET, B, 4), jnp.float32),
        ],
        scratch_shapes=[
            pltpu.VMEM((B, W), jnp.float32),
            pltpu.VMEM((B, W), jnp.float32),
        ],
    )(rest_f, eff_f)

    return s_t.T, c_t.T, b_t.transpose(1, 0, 2)


# speedup vs baseline: 10.1419x; 1.1146x over previous
import functools

import jax
import jax.numpy as jnp
from jax import lax
from jax.experimental import pallas as pl
from jax.experimental.pallas import tpu as pltpu
from jax.experimental.pallas import tpu_sc as plsc

IMAGE_W = 608
IMAGE_H = 608
TOP_N = 1000
MIN_SCORE = 0.05
NMS_THR = 0.5
MAX_DET = 100

_CHUNK = 2048


def _sig(x):
    return 1.0 / (1.0 + jnp.exp(-x))


def _decode_body(obj_ref, reg_ref, cls_ref, anch_ref, score_ref, rest_ref):
    o = obj_ref[0, 0, :, 0]
    creg = reg_ref[0, 0]
    ccls = cls_ref[0, 0]
    a = anch_ref[0, 0]
    m = jnp.max(ccls, axis=-1)
    amax = jnp.argmax(ccls, axis=-1).astype(jnp.float32)
    score = _sig(m) * _sig(o)
    stride = a[:, 4]
    xyx = (_sig(creg[:, 0]) + a[:, 0]) * stride
    xyy = (_sig(creg[:, 1]) + a[:, 1]) * stride
    whx = jnp.exp(creg[:, 2]) * a[:, 2] / stride
    why = jnp.exp(creg[:, 3]) * a[:, 3] / stride
    x1f = xyx - 0.5 * whx
    y1f = xyy - 0.5 * why
    x2f = whx + x1f
    y2f = why + y1f
    x1 = jnp.maximum(x1f.astype(jnp.int32), 0).astype(jnp.float32)
    y1 = jnp.maximum(y1f.astype(jnp.int32), 0).astype(jnp.float32)
    x2 = jnp.minimum(x2f.astype(jnp.int32), IMAGE_W - 1).astype(jnp.float32)
    y2 = jnp.minimum(y2f.astype(jnp.int32), IMAGE_H - 1).astype(jnp.float32)
    score_ref[0, 0, 0, :] = score
    rest_ref[0, 0, 0, 0, :] = amax
    rest_ref[1, 0, 0, 0, :] = x1
    rest_ref[2, 0, 0, 0, :] = y1
    rest_ref[3, 0, 0, 0, :] = x2
    rest_ref[4, 0, 0, 0, :] = y2


def _threshold_body(score_ref, eff_ref):
    score = score_ref[...]
    B, L, N = score.shape
    bits = jax.lax.bitcast_convert_type(score, jnp.int32)
    idx = jax.lax.broadcasted_iota(jnp.int32, (B, L, N), 2)

    def bit_step(_, carry):
        lo, hi = carry
        mid = jax.lax.shift_right_logical(lo + hi + 1, 1)
        cnt = jnp.sum((bits >= mid[:, :, None]).astype(jnp.int32), axis=-1)
        ok = cnt >= TOP_N
        return jnp.where(ok, mid, lo), jnp.where(ok, hi, mid - 1)

    lo0 = jnp.zeros((B, L), jnp.int32)
    hi0 = jnp.full((B, L), 0x3F800000, jnp.int32)
    t, _ = jax.lax.fori_loop(0, 31, bit_step, (lo0, hi0))
    gt = bits > t[:, :, None]
    eq = bits == t[:, :, None]
    need = TOP_N - jnp.sum(gt.astype(jnp.int32), axis=-1)

    def idx_step(_, carry):
        lo, hi = carry
        mid = jax.lax.shift_right_logical(lo + hi, 1)
        cnt = jnp.sum((eq & (idx < mid[:, :, None])).astype(jnp.int32), axis=-1)
        ok = cnt >= need
        return jnp.where(ok, lo, mid + 1), jnp.where(ok, mid, hi)

    lo0i = jnp.zeros((B, L), jnp.int32)
    hi0i = jnp.full((B, L), N, jnp.int32)
    _, icut = jax.lax.fori_loop(0, 15, idx_step, (lo0i, hi0i))
    mask = gt | (eq & (idx < icut[:, :, None]))
    eff_ref[...] = jnp.where(mask & (score > MIN_SCORE), score, -1.0)


def _nms_body(planes_ref, s_ref, c_ref, b_ref, scr_ref, ha_ref):
    _, B, W = planes_ref.shape
    cls_p = planes_ref[1]
    x1 = planes_ref[2]
    y1 = planes_ref[3]
    x2 = planes_ref[4]
    y2 = planes_ref[5]
    scr_ref[...] = planes_ref[0]
    ha_ref[...] = 0.5 * ((x2 - x1) * (y2 - y1))
    lane = jax.lax.broadcasted_iota(jnp.int32, (B, W), 1)

    def step(k, _):
        masked = scr_ref[...]
        m = jnp.max(masked, axis=1)
        sel = jnp.argmax(masked, axis=1)
        found = m > 0.0
        onehot = lane == sel[:, None]
        neg = jnp.float32(-1e30)
        sx1 = jnp.max(jnp.where(onehot, x1, neg), axis=1)
        sy1 = jnp.max(jnp.where(onehot, y1, neg), axis=1)
        sx2 = jnp.max(jnp.where(onehot, x2, neg), axis=1)
        sy2 = jnp.max(jnp.where(onehot, y2, neg), axis=1)
        sc = jnp.max(jnp.where(onehot, cls_p, neg), axis=1)
        s_ref[pl.ds(k, 1), :] = jnp.where(found, m, -1.0)[None, :]
        c_ref[pl.ds(k, 1), :] = jnp.where(found, sc, -1.0)[None, :]
        box = jnp.stack([sx1, sy1, sx2, sy2], axis=-1)
        b_ref[pl.ds(k, 1), :, :] = jnp.where(found[:, None], box, -1.0)[None]
        sarea_eps = (sx2 - sx1) * (sy2 - sy1) + jnp.float32(1e-9)
        xx1 = jnp.maximum(x1, sx1[:, None])
        yy1 = jnp.maximum(y1, sy1[:, None])
        xx2 = jnp.minimum(x2, sx2[:, None])
        yy2 = jnp.minimum(y2, sy2[:, None])
        iw = jnp.maximum(xx2 - xx1, 0.0)
        ih = jnp.maximum(yy2 - yy1, 0.0)
        inter = iw * ih
        rhs = ha_ref[...] + (0.5 * sarea_eps)[:, None]
        sup = 1.5 * inter > rhs
        kill = (sup | onehot) & found[:, None]
        scr_ref[...] = jnp.where(kill, -1.0, masked)
        return 0

    jax.lax.fori_loop(0, MAX_DET, step, 0)


_SEG = 1024


def _compact_sc(eff, rest, B, L, N):
    n_vecs = N // 16
    seg_vecs = _SEG // 16
    mesh = plsc.VectorSubcoreMesh(core_axis_name="c", subcore_axis_name="s")

    @functools.partial(
        pl.kernel,
        mesh=mesh,
        out_type=jax.ShapeDtypeStruct((6, B, L * _SEG), jnp.float32),
        scratch_types=(
            [pltpu.VMEM((N,), jnp.float32) for _ in range(6)]
            + [pltpu.VMEM((_SEG,), jnp.float32) for _ in range(6)]
        ),
        compiler_params=pltpu.CompilerParams(needs_layout_passes=False),
    )
    def body(eff_hbm, rest_hbm, out_hbm,
             ie, ic, ix1, iy1, ix2, iy2, oe, oc, ox1, oy1, ox2, oy2):
        wid = lax.axis_index("s") * 2 + lax.axis_index("c")

        @pl.when(wid < B * L)
        def _():
            b = wid // L
            l = wid % L
            pltpu.sync_copy(eff_hbm.at[b, l], ie)
            pltpu.sync_copy(rest_hbm.at[0, b, l], ic)
            pltpu.sync_copy(rest_hbm.at[1, b, l], ix1)
            pltpu.sync_copy(rest_hbm.at[2, b, l], iy1)
            pltpu.sync_copy(rest_hbm.at[3, b, l], ix2)
            pltpu.sync_copy(rest_hbm.at[4, b, l], iy2)
            neg1 = jnp.full((16,), -1.0, jnp.float32)

            def init(i, _):
                sl = pl.ds(i * 16, 16)
                oe[sl] = neg1
                oc[sl] = neg1
                ox1[sl] = neg1
                oy1[sl] = neg1
                ox2[sl] = neg1
                oy2[sl] = neg1
                return 0

            lax.fori_loop(0, seg_vecs, init, 0)

            def step(i, off):
                sl = pl.ds(i * 16, 16)
                e = ie[sl]
                mask = e > 0.0
                pc = jnp.cumsum(mask.astype(jnp.int32))
                dst = off + pc - 1
                plsc.store_scatter(oe, [dst], e, mask=mask)
                plsc.store_scatter(oc, [dst], ic[sl], mask=mask)
                plsc.store_scatter(ox1, [dst], ix1[sl], mask=mask)
                plsc.store_scatter(oy1, [dst], iy1[sl], mask=mask)
                plsc.store_scatter(ox2, [dst], ix2[sl], mask=mask)
                plsc.store_scatter(oy2, [dst], iy2[sl], mask=mask)
                return off + plsc.all_reduce_population_count(mask)

            lax.fori_loop(0, n_vecs, step, jnp.zeros((16,), jnp.int32))
            seg = pl.ds(l * _SEG, _SEG)
            pltpu.sync_copy(oe, out_hbm.at[0, b, seg])
            pltpu.sync_copy(oc, out_hbm.at[1, b, seg])
            pltpu.sync_copy(ox1, out_hbm.at[2, b, seg])
            pltpu.sync_copy(oy1, out_hbm.at[3, b, seg])
            pltpu.sync_copy(ox2, out_hbm.at[4, b, seg])
            pltpu.sync_copy(oy2, out_hbm.at[5, b, seg])

    return body(eff, rest)


def kernel(obj_heads, reg_heads, cls_heads, batch_anchors):
    L, B, N, C = cls_heads.shape
    W = L * N
    nc = N // _CHUNK

    score, rest = pl.pallas_call(
        _decode_body,
        grid=(L, B, nc),
        in_specs=[
            pl.BlockSpec((1, 1, _CHUNK, 1), lambda l, b, c: (l, b, c, 0)),
            pl.BlockSpec((1, 1, _CHUNK, 4), lambda l, b, c: (l, b, c, 0)),
            pl.BlockSpec((1, 1, _CHUNK, C), lambda l, b, c: (l, b, c, 0)),
            pl.BlockSpec((1, 1, _CHUNK, 5), lambda l, b, c: (l, b, c, 0)),
        ],
        out_specs=[
            pl.BlockSpec((1, 1, 1, _CHUNK), lambda l, b, c: (b, l, 0, c)),
            pl.BlockSpec((5, 1, 1, 1, _CHUNK), lambda l, b, c: (0, b, l, 0, c)),
        ],
        out_shape=[
            jax.ShapeDtypeStruct((B, L, 1, N), jnp.float32),
            jax.ShapeDtypeStruct((5, B, L, 1, N), jnp.float32),
        ],
    )(obj_heads, reg_heads, cls_heads, batch_anchors)

    eff = pl.pallas_call(
        _threshold_body,
        out_shape=jax.ShapeDtypeStruct((B, L, N), jnp.float32),
    )(score.reshape(B, L, N))

    comp = _compact_sc(eff, rest.reshape(5, B, L, N), B, L, N)
    Wc = L * _SEG

    s_t, c_t, b_t = pl.pallas_call(
        _nms_body,
        out_shape=[
            jax.ShapeDtypeStruct((MAX_DET, B), jnp.float32),
            jax.ShapeDtypeStruct((MAX_DET, B), jnp.float32),
            jax.ShapeDtypeStruct((MAX_DET, B, 4), jnp.float32),
        ],
        scratch_shapes=[
            pltpu.VMEM((B, Wc), jnp.float32),
            pltpu.VMEM((B, Wc), jnp.float32),
        ],
    )(comp)

    return s_t.T, c_t.T, b_t.transpose(1, 0, 2)


# speedup vs baseline: 59.6327x; 5.8798x over previous
import functools

import jax
import jax.numpy as jnp
from jax import lax
from jax.experimental import pallas as pl
from jax.experimental.pallas import tpu as pltpu
from jax.experimental.pallas import tpu_sc as plsc

IMAGE_W = 608
IMAGE_H = 608
TOP_N = 1000
MIN_SCORE = 0.05
NMS_THR = 0.5
MAX_DET = 100

_CHUNK = 2048


def _sig(x):
    return 1.0 / (1.0 + jnp.exp(-x))


def _decode_body(cls_ref, aux_ref, score_ref, rest_ref):
    ccls = cls_ref[0, 0]
    aux = aux_ref[0, 0]
    m = jnp.max(ccls, axis=0)
    amax = jnp.argmax(ccls, axis=0).astype(jnp.float32)
    score = _sig(m) * _sig(aux[9])
    stride = aux[8]
    xyx = (_sig(aux[0]) + aux[4]) * stride
    xyy = (_sig(aux[1]) + aux[5]) * stride
    whx = jnp.exp(aux[2]) * aux[6] / stride
    why = jnp.exp(aux[3]) * aux[7] / stride
    x1f = xyx - 0.5 * whx
    y1f = xyy - 0.5 * why
    x2f = whx + x1f
    y2f = why + y1f
    x1 = jnp.maximum(x1f.astype(jnp.int32), 0).astype(jnp.float32)
    y1 = jnp.maximum(y1f.astype(jnp.int32), 0).astype(jnp.float32)
    x2 = jnp.minimum(x2f.astype(jnp.int32), IMAGE_W - 1).astype(jnp.float32)
    y2 = jnp.minimum(y2f.astype(jnp.int32), IMAGE_H - 1).astype(jnp.float32)
    score_ref[0, 0, 0, :] = score
    rest_ref[0, 0, 0, 0, :] = amax
    rest_ref[1, 0, 0, 0, :] = x1
    rest_ref[2, 0, 0, 0, :] = y1
    rest_ref[3, 0, 0, 0, :] = x2
    rest_ref[4, 0, 0, 0, :] = y2


def _threshold_body(score_ref, eff_ref):
    score = score_ref[...]
    B, L, N = score.shape
    bits = jax.lax.bitcast_convert_type(score, jnp.int32)
    idx = jax.lax.broadcasted_iota(jnp.int32, (B, L, N), 2)

    def bit_step(_, carry):
        lo, hi = carry
        mid = jax.lax.shift_right_logical(lo + hi + 1, 1)
        cnt = jnp.sum((bits >= mid[:, :, None]).astype(jnp.int32), axis=-1)
        ok = cnt >= TOP_N
        return jnp.where(ok, mid, lo), jnp.where(ok, hi, mid - 1)

    lo0 = jnp.zeros((B, L), jnp.int32)
    hi0 = jnp.full((B, L), 0x3F800000, jnp.int32)
    t, _ = jax.lax.fori_loop(0, 31, bit_step, (lo0, hi0))
    gt = bits > t[:, :, None]
    eq = bits == t[:, :, None]
    need = TOP_N - jnp.sum(gt.astype(jnp.int32), axis=-1)

    def idx_step(_, carry):
        lo, hi = carry
        mid = jax.lax.shift_right_logical(lo + hi, 1)
        cnt = jnp.sum((eq & (idx < mid[:, :, None])).astype(jnp.int32), axis=-1)
        ok = cnt >= need
        return jnp.where(ok, lo, mid + 1), jnp.where(ok, mid, hi)

    lo0i = jnp.zeros((B, L), jnp.int32)
    hi0i = jnp.full((B, L), N, jnp.int32)
    _, icut = jax.lax.fori_loop(0, 15, idx_step, (lo0i, hi0i))
    mask = gt | (eq & (idx < icut[:, :, None]))
    eff_ref[...] = jnp.where(mask & (score > MIN_SCORE), score, -1.0)


def _nms_body(planes_ref, s_ref, c_ref, b_ref, scr_ref, ha_ref):
    _, B, W = planes_ref.shape
    cls_p = planes_ref[1]
    x1 = planes_ref[2]
    y1 = planes_ref[3]
    x2 = planes_ref[4]
    y2 = planes_ref[5]
    scr_ref[...] = planes_ref[0]
    ha_ref[...] = 0.5 * ((x2 - x1) * (y2 - y1))
    lane = jax.lax.broadcasted_iota(jnp.int32, (B, W), 1)

    def step(k, _):
        masked = scr_ref[...]
        m = jnp.max(masked, axis=1)
        sel = jnp.argmax(masked, axis=1)
        found = m > 0.0
        onehot = lane == sel[:, None]
        neg = jnp.float32(-1e30)
        sx1 = jnp.max(jnp.where(onehot, x1, neg), axis=1)
        sy1 = jnp.max(jnp.where(onehot, y1, neg), axis=1)
        sx2 = jnp.max(jnp.where(onehot, x2, neg), axis=1)
        sy2 = jnp.max(jnp.where(onehot, y2, neg), axis=1)
        sc = jnp.max(jnp.where(onehot, cls_p, neg), axis=1)
        s_ref[pl.ds(k, 1), :] = jnp.where(found, m, -1.0)[None, :]
        c_ref[pl.ds(k, 1), :] = jnp.where(found, sc, -1.0)[None, :]
        box = jnp.stack([sx1, sy1, sx2, sy2], axis=-1)
        b_ref[pl.ds(k, 1), :, :] = jnp.where(found[:, None], box, -1.0)[None]
        sarea_eps = (sx2 - sx1) * (sy2 - sy1) + jnp.float32(1e-9)
        xx1 = jnp.maximum(x1, sx1[:, None])
        yy1 = jnp.maximum(y1, sy1[:, None])
        xx2 = jnp.minimum(x2, sx2[:, None])
        yy2 = jnp.minimum(y2, sy2[:, None])
        iw = jnp.maximum(xx2 - xx1, 0.0)
        ih = jnp.maximum(yy2 - yy1, 0.0)
        inter = iw * ih
        rhs = ha_ref[...] + (0.5 * sarea_eps)[:, None]
        sup = 1.5 * inter > rhs
        kill = (sup | onehot) & found[:, None]
        scr_ref[...] = jnp.where(kill, -1.0, masked)
        return 0

    jax.lax.fori_loop(0, MAX_DET, step, 0)


_SEG = 1024


def _compact_sc(eff, rest, B, L, N):
    n_vecs = N // 16
    seg_vecs = _SEG // 16
    mesh = plsc.VectorSubcoreMesh(core_axis_name="c", subcore_axis_name="s")

    @functools.partial(
        pl.kernel,
        mesh=mesh,
        out_type=jax.ShapeDtypeStruct((6, B, L * _SEG), jnp.float32),
        scratch_types=(
            [pltpu.VMEM((N,), jnp.float32) for _ in range(6)]
            + [pltpu.VMEM((_SEG,), jnp.float32) for _ in range(6)]
        ),
        compiler_params=pltpu.CompilerParams(needs_layout_passes=False),
    )
    def body(eff_hbm, rest_hbm, out_hbm,
             ie, ic, ix1, iy1, ix2, iy2, oe, oc, ox1, oy1, ox2, oy2):
        wid = lax.axis_index("s") * 2 + lax.axis_index("c")

        @pl.when(wid < B * L)
        def _():
            b = wid // L
            l = wid % L
            pltpu.sync_copy(eff_hbm.at[b, l], ie)
            pltpu.sync_copy(rest_hbm.at[0, b, l], ic)
            pltpu.sync_copy(rest_hbm.at[1, b, l], ix1)
            pltpu.sync_copy(rest_hbm.at[2, b, l], iy1)
            pltpu.sync_copy(rest_hbm.at[3, b, l], ix2)
            pltpu.sync_copy(rest_hbm.at[4, b, l], iy2)
            neg1 = jnp.full((16,), -1.0, jnp.float32)

            def init(i, _):
                sl = pl.ds(i * 16, 16)
                oe[sl] = neg1
                oc[sl] = neg1
                ox1[sl] = neg1
                oy1[sl] = neg1
                ox2[sl] = neg1
                oy2[sl] = neg1
                return 0

            lax.fori_loop(0, seg_vecs, init, 0)

            def step(i, off):
                sl = pl.ds(i * 16, 16)
                e = ie[sl]
                mask = e > 0.0
                pc = jnp.cumsum(mask.astype(jnp.int32))
                dst = off + pc - 1
                plsc.store_scatter(oe, [dst], e, mask=mask)
                plsc.store_scatter(oc, [dst], ic[sl], mask=mask)
                plsc.store_scatter(ox1, [dst], ix1[sl], mask=mask)
                plsc.store_scatter(oy1, [dst], iy1[sl], mask=mask)
                plsc.store_scatter(ox2, [dst], ix2[sl], mask=mask)
                plsc.store_scatter(oy2, [dst], iy2[sl], mask=mask)
                return off + plsc.all_reduce_population_count(mask)

            lax.fori_loop(0, n_vecs, step, jnp.zeros((16,), jnp.int32))
            seg = pl.ds(l * _SEG, _SEG)
            pltpu.sync_copy(oe, out_hbm.at[0, b, seg])
            pltpu.sync_copy(oc, out_hbm.at[1, b, seg])
            pltpu.sync_copy(ox1, out_hbm.at[2, b, seg])
            pltpu.sync_copy(oy1, out_hbm.at[3, b, seg])
            pltpu.sync_copy(ox2, out_hbm.at[4, b, seg])
            pltpu.sync_copy(oy2, out_hbm.at[5, b, seg])

    return body(eff, rest)


def kernel(obj_heads, reg_heads, cls_heads, batch_anchors):
    L, B, N, C = cls_heads.shape
    W = L * N
    nc = N // _CHUNK

    cls_t = cls_heads.transpose(0, 1, 3, 2)
    aux = jnp.concatenate(
        [
            reg_heads.transpose(0, 1, 3, 2),
            batch_anchors.transpose(0, 1, 3, 2),
            obj_heads.transpose(0, 1, 3, 2),
        ],
        axis=2,
    )

    score, rest = pl.pallas_call(
        _decode_body,
        grid=(L, B, nc),
        in_specs=[
            pl.BlockSpec((1, 1, C, _CHUNK), lambda l, b, c: (l, b, 0, c)),
            pl.BlockSpec((1, 1, 10, _CHUNK), lambda l, b, c: (l, b, 0, c)),
        ],
        out_specs=[
            pl.BlockSpec((1, 1, 1, _CHUNK), lambda l, b, c: (b, l, 0, c)),
            pl.BlockSpec((5, 1, 1, 1, _CHUNK), lambda l, b, c: (0, b, l, 0, c)),
        ],
        out_shape=[
            jax.ShapeDtypeStruct((B, L, 1, N), jnp.float32),
            jax.ShapeDtypeStruct((5, B, L, 1, N), jnp.float32),
        ],
    )(cls_t, aux)

    eff = pl.pallas_call(
        _threshold_body,
        out_shape=jax.ShapeDtypeStruct((B, L, N), jnp.float32),
    )(score.reshape(B, L, N))

    comp = _compact_sc(eff, rest.reshape(5, B, L, N), B, L, N)
    Wc = L * _SEG

    s_t, c_t, b_t = pl.pallas_call(
        _nms_body,
        out_shape=[
            jax.ShapeDtypeStruct((MAX_DET, B), jnp.float32),
            jax.ShapeDtypeStruct((MAX_DET, B), jnp.float32),
            jax.ShapeDtypeStruct((MAX_DET, B, 4), jnp.float32),
        ],
        scratch_shapes=[
            pltpu.VMEM((B, Wc), jnp.float32),
            pltpu.VMEM((B, Wc), jnp.float32),
        ],
    )(comp)

    return s_t.T, c_t.T, b_t.transpose(1, 0, 2)


# speedup vs baseline: 68.6000x; 1.1504x over previous
import functools

import jax
import jax.numpy as jnp
from jax import lax
from jax.experimental import pallas as pl
from jax.experimental.pallas import tpu as pltpu
from jax.experimental.pallas import tpu_sc as plsc

IMAGE_W = 608
IMAGE_H = 608
TOP_N = 1000
MIN_SCORE = 0.05
NMS_THR = 0.5
MAX_DET = 100

_CHUNK = 4096


def _sig(x):
    return 1.0 / (1.0 + jnp.exp(-x))


def _decode_body(cls_ref, aux_ref, score_ref, rest_ref):
    ccls = cls_ref[0, 0]
    aux = aux_ref[0, 0]
    m = jnp.max(ccls, axis=0)
    amax = jnp.argmax(ccls, axis=0).astype(jnp.float32)
    score = _sig(m) * _sig(aux[9])
    stride = aux[8]
    xyx = (_sig(aux[0]) + aux[4]) * stride
    xyy = (_sig(aux[1]) + aux[5]) * stride
    whx = jnp.exp(aux[2]) * aux[6] / stride
    why = jnp.exp(aux[3]) * aux[7] / stride
    x1f = xyx - 0.5 * whx
    y1f = xyy - 0.5 * why
    x2f = whx + x1f
    y2f = why + y1f
    x1 = jnp.maximum(x1f.astype(jnp.int32), 0).astype(jnp.float32)
    y1 = jnp.maximum(y1f.astype(jnp.int32), 0).astype(jnp.float32)
    x2 = jnp.minimum(x2f.astype(jnp.int32), IMAGE_W - 1).astype(jnp.float32)
    y2 = jnp.minimum(y2f.astype(jnp.int32), IMAGE_H - 1).astype(jnp.float32)
    score_ref[0, 0, 0, :] = score
    rest_ref[0, 0, 0, 0, :] = amax
    rest_ref[1, 0, 0, 0, :] = x1
    rest_ref[2, 0, 0, 0, :] = y1
    rest_ref[3, 0, 0, 0, :] = x2
    rest_ref[4, 0, 0, 0, :] = y2


def _threshold_body(score_ref, eff_ref):
    score = score_ref[...]
    B, L, N = score.shape
    bits = jax.lax.bitcast_convert_type(score, jnp.int32)
    idx = jax.lax.broadcasted_iota(jnp.int32, (B, L, N), 2)

    def bit_step(_, carry):
        lo, hi = carry
        mid = jax.lax.shift_right_logical(lo + hi + 1, 1)
        cnt = jnp.sum((bits >= mid[:, :, None]).astype(jnp.int32), axis=-1)
        ok = cnt >= TOP_N
        return jnp.where(ok, mid, lo), jnp.where(ok, hi, mid - 1)

    lo0 = jnp.zeros((B, L), jnp.int32)
    hi0 = jnp.full((B, L), 0x3F800000, jnp.int32)
    t, _ = jax.lax.fori_loop(0, 31, bit_step, (lo0, hi0))
    gt = bits > t[:, :, None]
    eq = bits == t[:, :, None]
    need = TOP_N - jnp.sum(gt.astype(jnp.int32), axis=-1)

    def idx_step(_, carry):
        lo, hi = carry
        mid = jax.lax.shift_right_logical(lo + hi, 1)
        cnt = jnp.sum((eq & (idx < mid[:, :, None])).astype(jnp.int32), axis=-1)
        ok = cnt >= need
        return jnp.where(ok, lo, mid + 1), jnp.where(ok, mid, hi)

    lo0i = jnp.zeros((B, L), jnp.int32)
    hi0i = jnp.full((B, L), N, jnp.int32)
    _, icut = jax.lax.fori_loop(0, 15, idx_step, (lo0i, hi0i))
    mask = gt | (eq & (idx < icut[:, :, None]))
    eff_ref[...] = jnp.where(mask & (score > MIN_SCORE), score, -1.0)


def _nms_body(planes_ref, s_ref, c_ref, b_ref, scr_ref, ha_ref):
    _, B, W = planes_ref.shape
    cls_p = planes_ref[1]
    x1 = planes_ref[2]
    y1 = planes_ref[3]
    x2 = planes_ref[4]
    y2 = planes_ref[5]
    scr_ref[...] = planes_ref[0]
    ha_ref[...] = 0.5 * ((x2 - x1) * (y2 - y1))
    lane = jax.lax.broadcasted_iota(jnp.int32, (B, W), 1)

    def step(k, _):
        masked = scr_ref[...]
        m = jnp.max(masked, axis=1)
        sel = jnp.argmax(masked, axis=1)
        found = m > 0.0
        onehot = lane == sel[:, None]
        neg = jnp.float32(-1e30)
        sx1 = jnp.max(jnp.where(onehot, x1, neg), axis=1)
        sy1 = jnp.max(jnp.where(onehot, y1, neg), axis=1)
        sx2 = jnp.max(jnp.where(onehot, x2, neg), axis=1)
        sy2 = jnp.max(jnp.where(onehot, y2, neg), axis=1)
        sc = jnp.max(jnp.where(onehot, cls_p, neg), axis=1)
        s_ref[pl.ds(k, 1), :] = jnp.where(found, m, -1.0)[None, :]
        c_ref[pl.ds(k, 1), :] = jnp.where(found, sc, -1.0)[None, :]
        box = jnp.stack([sx1, sy1, sx2, sy2], axis=-1)
        b_ref[pl.ds(k, 1), :, :] = jnp.where(found[:, None], box, -1.0)[None]
        sarea_eps = (sx2 - sx1) * (sy2 - sy1) + jnp.float32(1e-9)
        xx1 = jnp.maximum(x1, sx1[:, None])
        yy1 = jnp.maximum(y1, sy1[:, None])
        xx2 = jnp.minimum(x2, sx2[:, None])
        yy2 = jnp.minimum(y2, sy2[:, None])
        iw = jnp.maximum(xx2 - xx1, 0.0)
        ih = jnp.maximum(yy2 - yy1, 0.0)
        inter = iw * ih
        rhs = ha_ref[...] + (0.5 * sarea_eps)[:, None]
        sup = 1.5 * inter > rhs
        kill = (sup | onehot) & found[:, None]
        scr_ref[...] = jnp.where(kill, -1.0, masked)
        return 0

    jax.lax.fori_loop(0, MAX_DET, step, 0)


_SEG = 1024


def _compact_sc(eff, rest, B, L, N):
    n_vecs = N // 16
    seg_vecs = _SEG // 16
    mesh = plsc.VectorSubcoreMesh(core_axis_name="c", subcore_axis_name="s")

    @functools.partial(
        pl.kernel,
        mesh=mesh,
        out_type=jax.ShapeDtypeStruct((6, B, L * _SEG), jnp.float32),
        scratch_types=(
            [pltpu.VMEM((N,), jnp.float32) for _ in range(6)]
            + [pltpu.VMEM((_SEG,), jnp.float32) for _ in range(6)]
        ),
        compiler_params=pltpu.CompilerParams(needs_layout_passes=False),
    )
    def body(eff_hbm, rest_hbm, out_hbm,
             ie, ic, ix1, iy1, ix2, iy2, oe, oc, ox1, oy1, ox2, oy2):
        wid = lax.axis_index("s") * 2 + lax.axis_index("c")

        @pl.when(wid < B * L)
        def _():
            b = wid // L
            l = wid % L
            pltpu.sync_copy(eff_hbm.at[b, l], ie)
            pltpu.sync_copy(rest_hbm.at[0, b, l], ic)
            pltpu.sync_copy(rest_hbm.at[1, b, l], ix1)
            pltpu.sync_copy(rest_hbm.at[2, b, l], iy1)
            pltpu.sync_copy(rest_hbm.at[3, b, l], ix2)
            pltpu.sync_copy(rest_hbm.at[4, b, l], iy2)
            neg1 = jnp.full((16,), -1.0, jnp.float32)

            def init(i, _):
                sl = pl.ds(i * 16, 16)
                oe[sl] = neg1
                oc[sl] = neg1
                ox1[sl] = neg1
                oy1[sl] = neg1
                ox2[sl] = neg1
                oy2[sl] = neg1
                return 0

            lax.fori_loop(0, seg_vecs, init, 0)

            def step(i, off):
                sl = pl.ds(i * 16, 16)
                e = ie[sl]
                mask = e > 0.0
                pc = jnp.cumsum(mask.astype(jnp.int32))
                dst = off + pc - 1
                plsc.store_scatter(oe, [dst], e, mask=mask)
                plsc.store_scatter(oc, [dst], ic[sl], mask=mask)
                plsc.store_scatter(ox1, [dst], ix1[sl], mask=mask)
                plsc.store_scatter(oy1, [dst], iy1[sl], mask=mask)
                plsc.store_scatter(ox2, [dst], ix2[sl], mask=mask)
                plsc.store_scatter(oy2, [dst], iy2[sl], mask=mask)
                return off + plsc.all_reduce_population_count(mask)

            lax.fori_loop(0, n_vecs, step, jnp.zeros((16,), jnp.int32))
            seg = pl.ds(l * _SEG, _SEG)
            pltpu.sync_copy(oe, out_hbm.at[0, b, seg])
            pltpu.sync_copy(oc, out_hbm.at[1, b, seg])
            pltpu.sync_copy(ox1, out_hbm.at[2, b, seg])
            pltpu.sync_copy(oy1, out_hbm.at[3, b, seg])
            pltpu.sync_copy(ox2, out_hbm.at[4, b, seg])
            pltpu.sync_copy(oy2, out_hbm.at[5, b, seg])

    return body(eff, rest)


def kernel(obj_heads, reg_heads, cls_heads, batch_anchors):
    L, B, N, C = cls_heads.shape
    W = L * N
    nc = N // _CHUNK

    cls_t = cls_heads.transpose(0, 1, 3, 2)
    aux = jnp.concatenate(
        [
            reg_heads.transpose(0, 1, 3, 2),
            batch_anchors.transpose(0, 1, 3, 2),
            obj_heads.transpose(0, 1, 3, 2),
        ],
        axis=2,
    )

    score, rest = pl.pallas_call(
        _decode_body,
        grid=(L, B, nc),
        in_specs=[
            pl.BlockSpec((1, 1, C, _CHUNK), lambda l, b, c: (l, b, 0, c)),
            pl.BlockSpec((1, 1, 10, _CHUNK), lambda l, b, c: (l, b, 0, c)),
        ],
        out_specs=[
            pl.BlockSpec((1, 1, 1, _CHUNK), lambda l, b, c: (b, l, 0, c)),
            pl.BlockSpec((5, 1, 1, 1, _CHUNK), lambda l, b, c: (0, b, l, 0, c)),
        ],
        out_shape=[
            jax.ShapeDtypeStruct((B, L, 1, N), jnp.float32),
            jax.ShapeDtypeStruct((5, B, L, 1, N), jnp.float32),
        ],
    )(cls_t, aux)

    eff = pl.pallas_call(
        _threshold_body,
        out_shape=jax.ShapeDtypeStruct((B, L, N), jnp.float32),
    )(score.reshape(B, L, N))

    comp = _compact_sc(eff, rest.reshape(5, B, L, N), B, L, N)
    Wc = L * _SEG

    s_t, c_t, b_t = pl.pallas_call(
        _nms_body,
        out_shape=[
            jax.ShapeDtypeStruct((MAX_DET, B), jnp.float32),
            jax.ShapeDtypeStruct((MAX_DET, B), jnp.float32),
            jax.ShapeDtypeStruct((MAX_DET, B, 4), jnp.float32),
        ],
        scratch_shapes=[
            pltpu.VMEM((B, Wc), jnp.float32),
            pltpu.VMEM((B, Wc), jnp.float32),
        ],
    )(comp)

    return s_t.T, c_t.T, b_t.transpose(1, 0, 2)


# speedup vs baseline: 73.9276x; 1.0777x over previous
import functools

import jax
import jax.numpy as jnp
from jax import lax
from jax.experimental import pallas as pl
from jax.experimental.pallas import tpu as pltpu
from jax.experimental.pallas import tpu_sc as plsc

IMAGE_W = 608
IMAGE_H = 608
TOP_N = 1000
MIN_SCORE = 0.05
NMS_THR = 0.5
MAX_DET = 100

_CHUNK = 8192


def _sig(x):
    return 1.0 / (1.0 + jnp.exp(-x))


def _decode_body(cls_ref, aux_ref, score_ref, rest_ref):
    ccls = cls_ref[0, 0]
    aux = aux_ref[0, 0]
    m = jnp.max(ccls, axis=0)
    amax = jnp.argmax(ccls, axis=0).astype(jnp.float32)
    score = _sig(m) * _sig(aux[9])
    stride = aux[8]
    xyx = (_sig(aux[0]) + aux[4]) * stride
    xyy = (_sig(aux[1]) + aux[5]) * stride
    whx = jnp.exp(aux[2]) * aux[6] / stride
    why = jnp.exp(aux[3]) * aux[7] / stride
    x1f = xyx - 0.5 * whx
    y1f = xyy - 0.5 * why
    x2f = whx + x1f
    y2f = why + y1f
    x1 = jnp.maximum(x1f.astype(jnp.int32), 0).astype(jnp.float32)
    y1 = jnp.maximum(y1f.astype(jnp.int32), 0).astype(jnp.float32)
    x2 = jnp.minimum(x2f.astype(jnp.int32), IMAGE_W - 1).astype(jnp.float32)
    y2 = jnp.minimum(y2f.astype(jnp.int32), IMAGE_H - 1).astype(jnp.float32)
    score_ref[0, 0, 0, :] = score
    rest_ref[0, 0, 0, 0, :] = amax
    rest_ref[1, 0, 0, 0, :] = x1
    rest_ref[2, 0, 0, 0, :] = y1
    rest_ref[3, 0, 0, 0, :] = x2
    rest_ref[4, 0, 0, 0, :] = y2


def _threshold_body(score_ref, eff_ref):
    score = score_ref[...]
    B, L, N = score.shape
    bits = jax.lax.bitcast_convert_type(score, jnp.int32)
    idx = jax.lax.broadcasted_iota(jnp.int32, (B, L, N), 2)

    def bit_step(_, carry):
        lo, hi = carry
        mid = jax.lax.shift_right_logical(lo + hi + 1, 1)
        cnt = jnp.sum((bits >= mid[:, :, None]).astype(jnp.int32), axis=-1)
        ok = cnt >= TOP_N
        return jnp.where(ok, mid, lo), jnp.where(ok, hi, mid - 1)

    lo0 = jnp.zeros((B, L), jnp.int32)
    hi0 = jnp.full((B, L), 0x3F800000, jnp.int32)
    t, _ = jax.lax.fori_loop(0, 31, bit_step, (lo0, hi0))
    gt = bits > t[:, :, None]
    eq = bits == t[:, :, None]
    need = TOP_N - jnp.sum(gt.astype(jnp.int32), axis=-1)

    def idx_step(_, carry):
        lo, hi = carry
        mid = jax.lax.shift_right_logical(lo + hi, 1)
        cnt = jnp.sum((eq & (idx < mid[:, :, None])).astype(jnp.int32), axis=-1)
        ok = cnt >= need
        return jnp.where(ok, lo, mid + 1), jnp.where(ok, mid, hi)

    lo0i = jnp.zeros((B, L), jnp.int32)
    hi0i = jnp.full((B, L), N, jnp.int32)
    _, icut = jax.lax.fori_loop(0, 15, idx_step, (lo0i, hi0i))
    mask = gt | (eq & (idx < icut[:, :, None]))
    eff_ref[...] = jnp.where(mask & (score > MIN_SCORE), score, -1.0)


def _nms_body(planes_ref, s_ref, c_ref, b_ref, scr_ref, ha_ref):
    _, B, W = planes_ref.shape
    cls_p = planes_ref[1]
    x1 = planes_ref[2]
    y1 = planes_ref[3]
    x2 = planes_ref[4]
    y2 = planes_ref[5]
    scr_ref[...] = planes_ref[0]
    ha_ref[...] = 0.5 * ((x2 - x1) * (y2 - y1))
    lane = jax.lax.broadcasted_iota(jnp.int32, (B, W), 1)

    def step(k, _):
        masked = scr_ref[...]
        m = jnp.max(masked, axis=1)
        sel = jnp.argmax(masked, axis=1)
        found = m > 0.0
        onehot = lane == sel[:, None]
        neg = jnp.float32(-1e30)
        sx1 = jnp.max(jnp.where(onehot, x1, neg), axis=1)
        sy1 = jnp.max(jnp.where(onehot, y1, neg), axis=1)
        sx2 = jnp.max(jnp.where(onehot, x2, neg), axis=1)
        sy2 = jnp.max(jnp.where(onehot, y2, neg), axis=1)
        sc = jnp.max(jnp.where(onehot, cls_p, neg), axis=1)
        s_ref[pl.ds(k, 1), :] = jnp.where(found, m, -1.0)[None, :]
        c_ref[pl.ds(k, 1), :] = jnp.where(found, sc, -1.0)[None, :]
        box = jnp.stack([sx1, sy1, sx2, sy2], axis=-1)
        b_ref[pl.ds(k, 1), :, :] = jnp.where(found[:, None], box, -1.0)[None]
        sarea_eps = (sx2 - sx1) * (sy2 - sy1) + jnp.float32(1e-9)
        xx1 = jnp.maximum(x1, sx1[:, None])
        yy1 = jnp.maximum(y1, sy1[:, None])
        xx2 = jnp.minimum(x2, sx2[:, None])
        yy2 = jnp.minimum(y2, sy2[:, None])
        iw = jnp.maximum(xx2 - xx1, 0.0)
        ih = jnp.maximum(yy2 - yy1, 0.0)
        inter = iw * ih
        rhs = ha_ref[...] + (0.5 * sarea_eps)[:, None]
        sup = 1.5 * inter > rhs
        kill = (sup | onehot) & found[:, None]
        scr_ref[...] = jnp.where(kill, -1.0, masked)
        return 0

    jax.lax.fori_loop(0, MAX_DET, step, 0)


_SEG = 1024


def _compact_sc(eff, rest, B, L, N):
    n_vecs = N // 16
    seg_vecs = _SEG // 16
    mesh = plsc.VectorSubcoreMesh(core_axis_name="c", subcore_axis_name="s")

    @functools.partial(
        pl.kernel,
        mesh=mesh,
        out_type=jax.ShapeDtypeStruct((6, B, L * _SEG), jnp.float32),
        scratch_types=(
            [pltpu.VMEM((N,), jnp.float32) for _ in range(6)]
            + [pltpu.VMEM((_SEG,), jnp.float32) for _ in range(6)]
        ),
        compiler_params=pltpu.CompilerParams(needs_layout_passes=False),
    )
    def body(eff_hbm, rest_hbm, out_hbm,
             ie, ic, ix1, iy1, ix2, iy2, oe, oc, ox1, oy1, ox2, oy2):
        wid = lax.axis_index("s") * 2 + lax.axis_index("c")

        @pl.when(wid < B * L)
        def _():
            b = wid // L
            l = wid % L
            pltpu.sync_copy(eff_hbm.at[b, l], ie)
            pltpu.sync_copy(rest_hbm.at[0, b, l], ic)
            pltpu.sync_copy(rest_hbm.at[1, b, l], ix1)
            pltpu.sync_copy(rest_hbm.at[2, b, l], iy1)
            pltpu.sync_copy(rest_hbm.at[3, b, l], ix2)
            pltpu.sync_copy(rest_hbm.at[4, b, l], iy2)
            neg1 = jnp.full((16,), -1.0, jnp.float32)

            def init(i, _):
                sl = pl.ds(i * 16, 16)
                oe[sl] = neg1
                oc[sl] = neg1
                ox1[sl] = neg1
                oy1[sl] = neg1
                ox2[sl] = neg1
                oy2[sl] = neg1
                return 0

            lax.fori_loop(0, seg_vecs, init, 0)

            def step(i, off):
                sl = pl.ds(i * 16, 16)
                e = ie[sl]
                mask = e > 0.0
                pc = jnp.cumsum(mask.astype(jnp.int32))
                dst = off + pc - 1
                plsc.store_scatter(oe, [dst], e, mask=mask)
                plsc.store_scatter(oc, [dst], ic[sl], mask=mask)
                plsc.store_scatter(ox1, [dst], ix1[sl], mask=mask)
                plsc.store_scatter(oy1, [dst], iy1[sl], mask=mask)
                plsc.store_scatter(ox2, [dst], ix2[sl], mask=mask)
                plsc.store_scatter(oy2, [dst], iy2[sl], mask=mask)
                return off + plsc.all_reduce_population_count(mask)

            lax.fori_loop(0, n_vecs, step, jnp.zeros((16,), jnp.int32))
            seg = pl.ds(l * _SEG, _SEG)
            pltpu.sync_copy(oe, out_hbm.at[0, b, seg])
            pltpu.sync_copy(oc, out_hbm.at[1, b, seg])
            pltpu.sync_copy(ox1, out_hbm.at[2, b, seg])
            pltpu.sync_copy(oy1, out_hbm.at[3, b, seg])
            pltpu.sync_copy(ox2, out_hbm.at[4, b, seg])
            pltpu.sync_copy(oy2, out_hbm.at[5, b, seg])

    return body(eff, rest)


def kernel(obj_heads, reg_heads, cls_heads, batch_anchors):
    L, B, N, C = cls_heads.shape
    W = L * N
    nc = N // _CHUNK

    cls_t = cls_heads.transpose(0, 1, 3, 2)
    aux = jnp.concatenate(
        [
            reg_heads.transpose(0, 1, 3, 2),
            batch_anchors.transpose(0, 1, 3, 2),
            obj_heads.transpose(0, 1, 3, 2),
        ],
        axis=2,
    )

    score, rest = pl.pallas_call(
        _decode_body,
        grid=(L, B, nc),
        in_specs=[
            pl.BlockSpec((1, 1, C, _CHUNK), lambda l, b, c: (l, b, 0, c)),
            pl.BlockSpec((1, 1, 10, _CHUNK), lambda l, b, c: (l, b, 0, c)),
        ],
        out_specs=[
            pl.BlockSpec((1, 1, 1, _CHUNK), lambda l, b, c: (b, l, 0, c)),
            pl.BlockSpec((5, 1, 1, 1, _CHUNK), lambda l, b, c: (0, b, l, 0, c)),
        ],
        out_shape=[
            jax.ShapeDtypeStruct((B, L, 1, N), jnp.float32),
            jax.ShapeDtypeStruct((5, B, L, 1, N), jnp.float32),
        ],
    )(cls_t, aux)

    eff = pl.pallas_call(
        _threshold_body,
        out_shape=jax.ShapeDtypeStruct((B, L, N), jnp.float32),
    )(score.reshape(B, L, N))

    comp = _compact_sc(eff, rest.reshape(5, B, L, N), B, L, N)
    Wc = L * _SEG

    s_t, c_t, b_t = pl.pallas_call(
        _nms_body,
        out_shape=[
            jax.ShapeDtypeStruct((MAX_DET, B), jnp.float32),
            jax.ShapeDtypeStruct((MAX_DET, B), jnp.float32),
            jax.ShapeDtypeStruct((MAX_DET, B, 4), jnp.float32),
        ],
        scratch_shapes=[
            pltpu.VMEM((B, Wc), jnp.float32),
            pltpu.VMEM((B, Wc), jnp.float32),
        ],
    )(comp)

    return s_t.T, c_t.T, b_t.transpose(1, 0, 2)
